# baseline (speedup 1.0000x reference)
import functools

import jax
import jax.numpy as jnp
from jax import lax
from jax.experimental import pallas as pl
from jax.experimental.pallas import tpu as pltpu
from jax.experimental.pallas import tpu_sc as plsc

NC = 2
NS = 16
L = 16
NW = NC * NS
B = 16384
F = 64
BPW = B // NW
CHUNK = 128
NCHUNK = BPW // CHUNK

_mesh = plsc.VectorSubcoreMesh(core_axis_name="c", subcore_axis_name="s")


@functools.partial(
    pl.kernel,
    out_type=jax.ShapeDtypeStruct((B,), jnp.float32),
    mesh=_mesh,
    compiler_params=pltpu.CompilerParams(
        needs_layout_passes=False, use_tc_tiling_on_sc=False),
    scratch_types=[
        pltpu.VMEM((NCHUNK, CHUNK), jnp.int32),
        pltpu.VMEM((NCHUNK, CHUNK), jnp.int32),
        pltpu.VMEM((2, CHUNK, F), jnp.float32),
        pltpu.VMEM((2, CHUNK, F), jnp.float32),
        pltpu.VMEM((F, L), jnp.float32),
        pltpu.VMEM((L,), jnp.float32),
        pltpu.VMEM((BPW,), jnp.float32),
        pltpu.SemaphoreType.DMA,
        pltpu.SemaphoreType.DMA,
    ],
)
def _gmf_sc(user_hbm, item_hbm, ut_hbm, it_hbm, wb_hbm, bb_hbm, out_hbm,
            uidx_v, iidx_v, urows_v, irows_v, w_v, b_v, out_v, sem0, sem1):
    wid = lax.axis_index("s") * NC + lax.axis_index("c")
    base = wid * BPW

    pltpu.sync_copy(user_hbm.at[pl.ds(wid * NCHUNK, NCHUNK)], uidx_v)
    pltpu.sync_copy(item_hbm.at[pl.ds(wid * NCHUNK, NCHUNK)], iidx_v)
    pltpu.sync_copy(wb_hbm, w_v)
    pltpu.sync_copy(bb_hbm, b_v)

    sems = [sem0, sem1]

    def start(c):
        slot = c % 2
        pltpu.async_copy(ut_hbm.at[uidx_v.at[c]], urows_v.at[slot], sems[slot])
        pltpu.async_copy(it_hbm.at[iidx_v.at[c]], irows_v.at[slot], sems[slot])

    def drain(c):
        slot = c % 2
        pltpu.make_async_copy(ut_hbm.at[uidx_v.at[c]], urows_v.at[slot], sems[slot]).wait()
        pltpu.make_async_copy(it_hbm.at[iidx_v.at[c]], irows_v.at[slot], sems[slot]).wait()

    start(0)
    start(1)

    bias = b_v[...]

    for c in range(NCHUNK):
        slot = c % 2
        drain(c)
        ub = urows_v.at[slot]
        ib = irows_v.at[slot]

        @plsc.parallel_loop(0, CHUNK // L, 1, unroll=2)
        def _group(g):
            row = g * L + lax.iota(jnp.int32, L)
            acc = bias
            for f in range(F):
                col = jnp.full((L,), f, jnp.int32)
                uf = plsc.load_gather(ub, [row, col])
                vf = plsc.load_gather(ib, [row, col])
                acc = acc + uf * vf * w_v[f, :]
            out_v[pl.ds(c * CHUNK + g * L, L)] = 1.0 / (1.0 + jnp.exp(-acc))

        if c + 2 < NCHUNK:
            start(c + 2)

    pltpu.sync_copy(out_v, out_hbm.at[pl.ds(base, BPW)])


def kernel(user, item, user_table, item_table, W, b):
    wb = jnp.broadcast_to(W.reshape(F, 1), (F, L))
    bb = jnp.broadcast_to(b.reshape(1), (L,))
    return _gmf_sc(user.reshape(B // CHUNK, CHUNK), item.reshape(B // CHUNK, CHUNK),
                   user_table, item_table, wb, bb)


# speedup vs baseline: 1.1536x; 1.1536x over previous
import functools

import jax
import jax.numpy as jnp
from jax import lax
from jax.experimental import pallas as pl
from jax.experimental.pallas import tpu as pltpu
from jax.experimental.pallas import tpu_sc as plsc

NC = 2
NS = 16
L = 16
NW = NC * NS
B = 16384
F = 64
N_ROWS = 100000
KOFF = 50176
BPW = B // NW
CHUNK = 128
NCHUNK = BPW // CHUNK

TBLK = 512
TGRID = KOFF // TBLK


def _pack_body(utl_ref, uth_ref, itl_ref, ith_ref, w_ref, up_ref, ip_ref):
    w = w_ref[...]
    ul = (utl_ref[...] * w).T
    uh = (uth_ref[...] * w).T
    up_ref[...] = jnp.concatenate([ul, uh], axis=1)
    ip_ref[...] = jnp.concatenate([itl_ref[...].T, ith_ref[...].T], axis=1)


_pack_tc = pl.pallas_call(
    _pack_body,
    grid=(TGRID,),
    in_specs=[
        pl.BlockSpec((F, TBLK), lambda b: (0, b)),
        pl.BlockSpec((F, TBLK), lambda b: (0, b + TGRID)),
        pl.BlockSpec((F, TBLK), lambda b: (0, b)),
        pl.BlockSpec((F, TBLK), lambda b: (0, b + TGRID)),
        pl.BlockSpec((F, 1), lambda b: (0, 0)),
    ],
    out_specs=[
        pl.BlockSpec((TBLK, 2 * F), lambda b: (b, 0)),
        pl.BlockSpec((TBLK, 2 * F), lambda b: (b, 0)),
    ],
    out_shape=[
        jax.ShapeDtypeStruct((KOFF, 2 * F), jnp.float32),
        jax.ShapeDtypeStruct((KOFF, 2 * F), jnp.float32),
    ],
)

_mesh = plsc.VectorSubcoreMesh(core_axis_name="c", subcore_axis_name="s")


@functools.partial(
    pl.kernel,
    out_type=jax.ShapeDtypeStruct((B,), jnp.float32),
    mesh=_mesh,
    compiler_params=pltpu.CompilerParams(needs_layout_passes=False),
    scratch_types=[
        pltpu.VMEM((NCHUNK, CHUNK), jnp.int32),
        pltpu.VMEM((NCHUNK, CHUNK), jnp.int32),
        pltpu.VMEM((NCHUNK, CHUNK), jnp.int32),
        pltpu.VMEM((NCHUNK, CHUNK), jnp.int32),
        pltpu.VMEM((2, CHUNK, 2 * F), jnp.float32),
        pltpu.VMEM((2, CHUNK, 2 * F), jnp.float32),
        pltpu.VMEM((L,), jnp.float32),
        pltpu.VMEM((BPW,), jnp.float32),
        pltpu.SemaphoreType.DMA,
        pltpu.SemaphoreType.DMA,
    ],
)
def _gmf_sc(user_hbm, item_hbm, up_hbm, ip_hbm, bb_hbm, out_hbm,
            uidx_v, iidx_v, updx_v, ipdx_v, urows_v, irows_v, b_v, out_v,
            sem0, sem1):
    wid = lax.axis_index("s") * NC + lax.axis_index("c")
    base = wid * BPW

    pltpu.sync_copy(user_hbm.at[pl.ds(wid * NCHUNK, NCHUNK)], uidx_v)
    pltpu.sync_copy(item_hbm.at[pl.ds(wid * NCHUNK, NCHUNK)], iidx_v)
    pltpu.sync_copy(bb_hbm, b_v)

    npair = jnp.full((L,), KOFF, jnp.int32)
    for c in range(NCHUNK):
        for g in range(CHUNK // L):
            s = pl.ds(g * L, L)
            u = uidx_v[c, s]
            i = iidx_v[c, s]
            updx_v[c, s] = jnp.where(u >= npair, u - npair, u)
            ipdx_v[c, s] = jnp.where(i >= npair, i - npair, i)

    sems = [sem0, sem1]

    def start(c):
        slot = c % 2
        pltpu.async_copy(up_hbm.at[updx_v.at[c]], urows_v.at[slot], sems[slot])
        pltpu.async_copy(ip_hbm.at[ipdx_v.at[c]], irows_v.at[slot], sems[slot])

    def drain(c):
        slot = c % 2
        pltpu.make_async_copy(up_hbm.at[updx_v.at[c]], urows_v.at[slot], sems[slot]).wait()
        pltpu.make_async_copy(ip_hbm.at[ipdx_v.at[c]], irows_v.at[slot], sems[slot]).wait()

    start(0)
    start(1)

    bias = b_v[...]
    zero = jnp.zeros((L,), jnp.int32)
    half = jnp.full((L,), F, jnp.int32)

    for c in range(NCHUNK):
        slot = c % 2
        drain(c)
        ub = urows_v.at[slot]
        ib = irows_v.at[slot]

        @plsc.parallel_loop(0, CHUNK // L, 1, unroll=2)
        def _group(g):
            s = pl.ds(g * L, L)
            row = g * L + lax.iota(jnp.int32, L)
            npair_v = jnp.full((L,), KOFF, jnp.int32)
            upar = jnp.where(uidx_v[c, s] >= npair_v, half, zero)
            ipar = jnp.where(iidx_v[c, s] >= npair_v, half, zero)
            acc = bias
            for f in range(F):
                uf = plsc.load_gather(ub, [row, upar + f])
                vf = plsc.load_gather(ib, [row, ipar + f])
                acc = acc + uf * vf
            out_v[pl.ds(c * CHUNK + g * L, L)] = 1.0 / (1.0 + jnp.exp(-acc))

        if c + 2 < NCHUNK:
            start(c + 2)

    pltpu.sync_copy(out_v, out_hbm.at[pl.ds(base, BPW)])


def kernel(user, item, user_table, item_table, W, b):
    ut = user_table.T
    it = item_table.T
    up, ip = _pack_tc(ut, ut, it, it, W.reshape(F, 1))
    bb = jnp.broadcast_to(b.reshape(1), (L,))
    return _gmf_sc(user.reshape(B // CHUNK, CHUNK), item.reshape(B // CHUNK, CHUNK),
                   up, ip, bb)
